# baseline (speedup 1.0000x reference)
import functools

import numpy as np
import jax
import jax.numpy as jnp
from jax.experimental import pallas as pl
from jax.experimental.pallas import tpu as pltpu

NB = 256
ROWPAD = 1024


def _conv1_map(off):
    rho = np.arange(256)[:, None] // 32
    v = np.arange(256)[:, None] % 32
    gw = np.arange(64)[None, :]
    dh = gw // 32
    dw = (gw // 16) % 2
    w15 = gw % 16
    i1 = rho - off - dh
    j1 = v - 2 * w15 - dw
    m1 = (i1 >= 0) & (i1 < 3) & (j1 >= 0) & (j1 < 3) & (w15 < 15)
    t_idx = np.where(m1, i1 * 3 + j1, 0).astype(np.int32)
    return t_idx, m1


def _build_static_maps():
    t_maps = [_conv1_map(off) for off in (0, 2, 4)]

    kappa = np.arange(8192)[:, None]
    rho2 = kappa // ROWPAD
    wc = (kappa % ROWPAD) // 64
    c = kappa % 64
    pg = np.arange(16)[None, :]
    par = pg // 8
    ow = pg % 8
    i2 = rho2 - par
    j2 = wc - ow
    m2 = (i2 >= 0) & (i2 < 7) & (j2 >= 0) & (j2 < 7)
    w_idx = np.where(m2, i2 * 448 + j2 * 64 + c, 0).astype(np.int32)

    l = np.arange(256)
    perm = (l % 16) * 16 + (l // 64) * 4 + ((l % 64) // 16)
    return t_maps, w_idx, m2, perm.astype(np.int32)


_T_MAPS, _W_IDX, _W_MASK, _FC1_PERM = _build_static_maps()


def _lenet_kernel(x_ref, ta_ref, tb_ref, tc_ref, w2_ref, b1_ref, b2_ref,
                  f1w_ref, f1b_ref, f2w_ref, f2b_ref, o_ref, y1s):
    xb = x_ref[...].astype(jnp.bfloat16)
    t_mats = [ta_ref[...], tb_ref[...], tc_ref[...]]
    b1 = b1_ref[...]
    for r in range(15):
        s = min(r // 2, 6)
        xs = xb[:, 128 * s:128 * s + 256]
        z = jnp.dot(xs, t_mats[r - 2 * s],
                    preferred_element_type=jnp.float32)
        m = jnp.maximum(jnp.maximum(z[:, :1024], z[:, 1024:2048]),
                        jnp.maximum(z[:, 2048:3072], z[:, 3072:4096]))
        y = jnp.tanh(m + b1)
        y1s[:, ROWPAD * r:ROWPAD * (r + 1)] = y.astype(jnp.bfloat16)

    w2 = w2_ref[...]
    feats = []
    for p in range(4):
        zp = jnp.dot(y1s[:, 2 * ROWPAD * p:2 * ROWPAD * p + 8192], w2,
                     preferred_element_type=jnp.float32)
        vp = jnp.maximum(zp[:, :128], zp[:, 128:])
        feats.extend(
            jnp.maximum(vp[:, 32 * q:32 * q + 16], vp[:, 32 * q + 16:32 * q + 32])
            for q in range(4))
    feat = jnp.tanh(jnp.concatenate(feats, axis=1) + b2_ref[...])

    h = jnp.tanh(
        jnp.dot(feat.astype(jnp.bfloat16), f1w_ref[...],
                preferred_element_type=jnp.float32) + f1b_ref[...])
    z2 = jnp.dot(h.astype(jnp.bfloat16), f2w_ref[...],
                 preferred_element_type=jnp.float32) + f2b_ref[...]
    mx = jnp.max(z2, axis=1, keepdims=True)
    s = jnp.sum(jnp.exp(z2 - mx), axis=1, keepdims=True)
    o_ref[...] = z2 - mx - jnp.log(s)


@jax.jit
def _forward(x, w1m, b1, w2r, b2, fc1_wt, fc1_b, fc2_wt, fc2_b):
    x2d = x.reshape(-1, 1024).astype(jnp.float32)
    B = x2d.shape[0]
    Bp = (B + NB - 1) // NB * NB
    if Bp != B:
        x2d = jnp.pad(x2d, ((0, Bp - B), (0, 0)))

    t_mats = [
        jnp.where(msk[:, :, None], w1m[idx], 0.0)
        .reshape(256, 4096).astype(jnp.bfloat16)
        for idx, msk in _T_MAPS
    ]
    w2f = w2r.reshape(3136, 16)
    w2t = jnp.where(_W_MASK[:, :, None], w2f[_W_IDX], 0.0)
    w2t = w2t.reshape(8192, 256).astype(jnp.bfloat16)
    b1t = jnp.pad(jnp.tile(b1.reshape(1, 64), (1, 15)),
                  ((0, 0), (0, 64)))
    b2t = jnp.tile(b2.reshape(1, 16), (1, 16))
    f1p = fc1_wt[_FC1_PERM].astype(jnp.bfloat16)
    f1b = fc1_b.reshape(1, 200)
    f2w = fc2_wt.astype(jnp.bfloat16)
    f2b = fc2_b.reshape(1, 10)

    out = pl.pallas_call(
        _lenet_kernel,
        out_shape=jax.ShapeDtypeStruct((Bp, 10), jnp.float32),
        grid=(Bp // NB,),
        in_specs=[
            pl.BlockSpec((NB, 1024), lambda b: (b, 0)),
            pl.BlockSpec((256, 4096), lambda b: (0, 0)),
            pl.BlockSpec((256, 4096), lambda b: (0, 0)),
            pl.BlockSpec((256, 4096), lambda b: (0, 0)),
            pl.BlockSpec((8192, 256), lambda b: (0, 0)),
            pl.BlockSpec((1, 1024), lambda b: (0, 0)),
            pl.BlockSpec((1, 256), lambda b: (0, 0)),
            pl.BlockSpec((256, 200), lambda b: (0, 0)),
            pl.BlockSpec((1, 200), lambda b: (0, 0)),
            pl.BlockSpec((200, 10), lambda b: (0, 0)),
            pl.BlockSpec((1, 10), lambda b: (0, 0)),
        ],
        out_specs=pl.BlockSpec((NB, 10), lambda b: (b, 0)),
        scratch_shapes=[pltpu.VMEM((NB, 15 * ROWPAD), jnp.bfloat16)],
        compiler_params=pltpu.CompilerParams(
            dimension_semantics=("parallel",),
            vmem_limit_bytes=100 * 1024 * 1024),
    )(x2d, t_mats[0], t_mats[1], t_mats[2], w2t, b1t, b2t, f1p, f1b, f2w, f2b)
    return out[:B]


def kernel(x, w1m, b1, w2r, b2, fc1_wt, fc1_b, fc2_wt, fc2_b):
    return _forward(x, w1m, b1, w2r, b2, fc1_wt, fc1_b, fc2_wt, fc2_b)


# speedup vs baseline: 1.9646x; 1.9646x over previous
import functools

import numpy as np
import jax
import jax.numpy as jnp
from jax.experimental import pallas as pl
from jax.experimental.pallas import tpu as pltpu

NB = 256
ROWPAD = 1024


_L = np.arange(256)
_FC1_PERM = ((_L % 16) * 16 + (_L // 64) * 4 + ((_L % 64) // 16)).astype(
    np.int32)


def _build_t_mats(w1m):
    pat = w1m.reshape(3, 3, 64)
    pat = jnp.pad(pat, ((0, 0), (0, 29), (0, 0)))
    pat = pat.reshape(96, 64)[:67]
    cols = []
    for g in range(4):
        dh, dw = g // 2, g % 2
        for w15 in range(16):
            if w15 == 15:
                cols.append(jnp.zeros((256, 64), w1m.dtype))
                continue
            off = dh * 32 + 2 * w15 + dw
            cols.append(jnp.pad(pat, ((off, 189 - off), (0, 0))))
    t0 = jnp.concatenate(cols, axis=1)
    t1 = jnp.pad(t0, ((64, 0), (0, 0)))[:256]
    t2 = jnp.pad(t0, ((128, 0), (0, 0)))[:256]
    return [t.astype(jnp.bfloat16) for t in (t0, t1, t2)]


def _build_w2t(w2r):
    pat = w2r.reshape(7, 7, 64, 16)
    pat = jnp.pad(pat, ((0, 0), (0, 9), (0, 0), (0, 0)))
    pat = pat.reshape(7 * 16 * 64, 16)[:6656]
    cols = []
    for pg in range(16):
        off = (pg // 8) * ROWPAD + (pg % 8) * 64
        cols.append(jnp.pad(pat, ((off, 1536 - off), (0, 0))))
    return jnp.concatenate(cols, axis=1).astype(jnp.bfloat16)


def _lenet_kernel(x_ref, ta_ref, tb_ref, tc_ref, w2_ref, b1_ref, b2_ref,
                  f1w_ref, f1b_ref, f2w_ref, f2b_ref, o_ref, y1s):
    xb = x_ref[...].astype(jnp.bfloat16)
    t_mats = [ta_ref[...], tb_ref[...], tc_ref[...]]
    b1 = b1_ref[...]
    for r in range(15):
        s = min(r // 2, 6)
        xs = xb[:, 128 * s:128 * s + 256]
        z = jnp.dot(xs, t_mats[r - 2 * s],
                    preferred_element_type=jnp.float32)
        m = jnp.maximum(jnp.maximum(z[:, :1024], z[:, 1024:2048]),
                        jnp.maximum(z[:, 2048:3072], z[:, 3072:4096]))
        y = jnp.tanh(m + b1)
        y1s[:, ROWPAD * r:ROWPAD * (r + 1)] = y.astype(jnp.bfloat16)

    w2 = w2_ref[...]
    feats = []
    for p in range(4):
        zp = jnp.dot(y1s[:, 2 * ROWPAD * p:2 * ROWPAD * p + 8192], w2,
                     preferred_element_type=jnp.float32)
        vp = jnp.maximum(zp[:, :128], zp[:, 128:])
        feats.extend(
            jnp.maximum(vp[:, 32 * q:32 * q + 16], vp[:, 32 * q + 16:32 * q + 32])
            for q in range(4))
    feat = jnp.tanh(jnp.concatenate(feats, axis=1) + b2_ref[...])

    h = jnp.tanh(
        jnp.dot(feat.astype(jnp.bfloat16), f1w_ref[...],
                preferred_element_type=jnp.float32) + f1b_ref[...])
    z2 = jnp.dot(h.astype(jnp.bfloat16), f2w_ref[...],
                 preferred_element_type=jnp.float32) + f2b_ref[...]
    mx = jnp.max(z2, axis=1, keepdims=True)
    s = jnp.sum(jnp.exp(z2 - mx), axis=1, keepdims=True)
    o_ref[...] = z2 - mx - jnp.log(s)


@jax.jit
def _forward(x, w1m, b1, w2r, b2, fc1_wt, fc1_b, fc2_wt, fc2_b):
    x2d = x.reshape(-1, 1024).astype(jnp.float32)
    B = x2d.shape[0]
    Bp = (B + NB - 1) // NB * NB
    if Bp != B:
        x2d = jnp.pad(x2d, ((0, Bp - B), (0, 0)))

    t_mats = _build_t_mats(w1m)
    w2t = _build_w2t(w2r)
    b1t = jnp.pad(jnp.tile(b1.reshape(1, 64), (1, 15)),
                  ((0, 0), (0, 64)))
    b2t = jnp.tile(b2.reshape(1, 16), (1, 16))
    f1p = fc1_wt[_FC1_PERM].astype(jnp.bfloat16)
    f1b = fc1_b.reshape(1, 200)
    f2w = fc2_wt.astype(jnp.bfloat16)
    f2b = fc2_b.reshape(1, 10)

    out = pl.pallas_call(
        _lenet_kernel,
        out_shape=jax.ShapeDtypeStruct((Bp, 10), jnp.float32),
        grid=(Bp // NB,),
        in_specs=[
            pl.BlockSpec((NB, 1024), lambda b: (b, 0)),
            pl.BlockSpec((256, 4096), lambda b: (0, 0)),
            pl.BlockSpec((256, 4096), lambda b: (0, 0)),
            pl.BlockSpec((256, 4096), lambda b: (0, 0)),
            pl.BlockSpec((8192, 256), lambda b: (0, 0)),
            pl.BlockSpec((1, 1024), lambda b: (0, 0)),
            pl.BlockSpec((1, 256), lambda b: (0, 0)),
            pl.BlockSpec((256, 200), lambda b: (0, 0)),
            pl.BlockSpec((1, 200), lambda b: (0, 0)),
            pl.BlockSpec((200, 10), lambda b: (0, 0)),
            pl.BlockSpec((1, 10), lambda b: (0, 0)),
        ],
        out_specs=pl.BlockSpec((NB, 10), lambda b: (b, 0)),
        scratch_shapes=[pltpu.VMEM((NB, 15 * ROWPAD), jnp.bfloat16)],
        compiler_params=pltpu.CompilerParams(
            dimension_semantics=("parallel",),
            vmem_limit_bytes=100 * 1024 * 1024),
    )(x2d, t_mats[0], t_mats[1], t_mats[2], w2t, b1t, b2t, f1p, f1b, f2w, f2b)
    return out[:B]


def kernel(x, w1m, b1, w2r, b2, fc1_wt, fc1_b, fc2_wt, fc2_b):
    return _forward(x, w1m, b1, w2r, b2, fc1_wt, fc1_b, fc2_wt, fc2_b)
